# baseline (speedup 1.0000x reference)
import functools

import jax
import jax.numpy as jnp
from jax import lax
from jax.experimental import pallas as pl
from jax.experimental.pallas import tpu as pltpu
from jax.experimental.pallas import tpu_sc as plsc

VOCAB = 49408
SEQ = 77
SEQP = 80
D = 768
BATCH = 4096
NC = 2
NS = 16
NW = NC * NS
SPW = BATCH // NW
C = 8
CPS = SEQP // C
TAIL = SEQ - C * (CPS - 1)
NBUF = 5
LEAD = 3
LANES = 16
DV = D // LANES


def _gather(tok_hbm, idx_ref, buf, gsem, slot):
    return pltpu.make_async_copy(tok_hbm.at[idx_ref], buf.at[slot],
                                 gsem.at[slot])


def _scatter(out_hbm, buf, ssem, row, c, slot):
    if c == CPS - 1:
        return pltpu.make_async_copy(
            buf.at[slot, pl.ds(0, TAIL)],
            out_hbm.at[row, pl.ds(c * C, TAIL)],
            ssem.at[slot])
    return pltpu.make_async_copy(
        buf.at[slot],
        out_hbm.at[row, pl.ds(c * C, C)],
        ssem.at[slot])


def _emb_kernel(idx_hbm, tok_hbm, pos_hbm, out_hbm, idx_v, pos_v, buf,
                gsem, ssem, isem):
    wid = lax.axis_index("s") * NC + lax.axis_index("c")
    wseq = wid * SPW

    pltpu.sync_copy(pos_hbm, pos_v)
    pltpu.sync_copy(idx_hbm.at[wseq], idx_v.at[0])

    for c in range(LEAD):
        _gather(tok_hbm, idx_v.at[0, pl.ds(c * C, C)], buf, gsem, c).start()

    @pl.loop(0, SPW)
    def seq_body(i):
        row = wseq + i
        sl = lax.rem(i, 2)
        sl2 = lax.rem(i + 1, 2)

        @pl.when(i + 1 < SPW)
        def _():
            pltpu.make_async_copy(idx_hbm.at[wseq + i + 1], idx_v.at[sl2],
                                  isem).start()

        for c in range(CPS):
            slot = c % NBUF
            _gather(tok_hbm, idx_v.at[sl, pl.ds(c * C, C)], buf, gsem,
                    slot).wait()
            nrows = TAIL if c == CPS - 1 else C

            @plsc.parallel_loop(0, nrows)
            def row_body(r):
                for d in range(DV):
                    plsc.addupdate(
                        buf.at[slot, r, pl.ds(d * LANES, LANES)],
                        pos_v[c * C + r, pl.ds(d * LANES, LANES)],
                    )
            _scatter(out_hbm, buf, ssem, row, c, slot).start()

            if c == CPS - LEAD - 1:
                @pl.when(i + 1 < SPW)
                def _():
                    pltpu.make_async_copy(idx_hbm.at[wseq + i + 1],
                                          idx_v.at[sl2], isem).wait()

            c2 = (c + LEAD) % CPS
            slot2 = c2 % NBUF
            cross = c + LEAD >= CPS
            isl = sl2 if cross else sl
            cond = (i + 1 < SPW) if cross else (i < SPW)

            @pl.when(cond)
            def _():
                g = i * CPS + c

                @pl.when(g + LEAD >= NBUF)
                def _():
                    cprev = (c + LEAD - NBUF) % CPS
                    rprev = row - (1 if c + LEAD - NBUF < 0 else 0)
                    _scatter(out_hbm, buf, ssem, rprev, cprev,
                             slot2).wait()
                _gather(tok_hbm, idx_v.at[isl, pl.ds(c2 * C, C)], buf,
                        gsem, slot2).start()

    for c in range(CPS - NBUF, CPS):
        _scatter(out_hbm, buf, ssem, wseq + SPW - 1, c, c % NBUF).wait()


@jax.jit
def _emb(idxp, token_table, pos_table):
    mesh = plsc.VectorSubcoreMesh(
        core_axis_name="c", subcore_axis_name="s", num_cores=NC, num_subcores=NS
    )
    f = functools.partial(
        pl.kernel,
        out_type=jax.ShapeDtypeStruct((BATCH, SEQ, D), jnp.float32),
        mesh=mesh,
        scratch_types=[
            pltpu.VMEM((2, SEQP), jnp.int32),
            pltpu.VMEM((SEQ, D), jnp.float32),
            pltpu.VMEM((NBUF, C, D), jnp.float32),
            pltpu.SemaphoreType.DMA((NBUF,)),
            pltpu.SemaphoreType.DMA((NBUF,)),
            pltpu.SemaphoreType.DMA,
        ],
    )(_emb_kernel)
    return f(idxp, token_table, pos_table)


def kernel(x, token_table, pos_table):
    idxp = jnp.pad(x.astype(jnp.int32), ((0, 0), (0, SEQP - SEQ)))
    return _emb(idxp, token_table, pos_table)
